# baseline (speedup 1.0000x reference)
import functools

import jax
import jax.numpy as jnp
from jax import lax
from jax.experimental import pallas as pl
from jax.experimental.pallas import tpu as pltpu
from jax.experimental.pallas import tpu_sc as plsc

H = W = 512
HW = H * W
D = 128
CELL = 16
BORDER = 16
NC = 30
N = NC * NC
NP = 1024
POS_R = 2

_OFFS = [(i, j) for i in range(-POS_R, POS_R + 1)
         for j in range(-POS_R, POS_R + 1) if i * i + j * j <= POS_R ** 2]
K = len(_OFFS)

NTILES = 32
SPT = NP // NTILES


def _sc_mesh():
    return plsc.VectorSubcoreMesh(core_axis_name="c", subcore_axis_name="s",
                                  num_cores=2, num_subcores=16)


def _colmax_kernel(d1_ref, d2_ref, out_ref):
    io = lax.broadcasted_iota(jnp.int32, (NC, CELL, NC * CELL), 1)
    res = []
    for dref in (d1_ref, d2_ref):
        x = dref[...].reshape(NC, CELL, NC * CELL)
        m = jnp.max(x, axis=1)
        r = jnp.min(jnp.where(x >= m[:, None, :], io, CELL), axis=1)
        res += [m, lax.bitcast_convert_type(r, jnp.float32)]
    out_ref[...] = jnp.stack(res, axis=0)


def _run_colmax(d1c, d2c):
    return pl.pallas_call(
        _colmax_kernel,
        out_shape=jax.ShapeDtypeStruct((4, NC, NC * CELL), jnp.float32),
    )(d1c, d2c)


def _argmax_kernel(m1_ref, r1_ref, m2_ref, r2_ref,
                   p1_ref, pd_ref, drow_ref, dcol_ref, tg1_ref):
    lane = lax.broadcasted_iota(jnp.int32, (1, N), 1)
    ci = lane // NC
    cj = lane % NC
    rjio = lax.broadcasted_iota(jnp.int32, (CELL, N), 0)

    def cell_argmax(m_ref, r_ref):
        m = m_ref[...]
        vmax = jnp.max(m, axis=0, keepdims=True)
        chan = r_ref[...] * CELL + rjio
        return jnp.min(jnp.where(m >= vmax, chan, 512), axis=0, keepdims=True)

    i1 = cell_argmax(m1_ref, r1_ref)
    i2 = cell_argmax(m2_ref, r2_ref)

    def coords(i):
        ri = i // CELL
        rj = i % CELL
        sx = jnp.clip(BORDER + cj * CELL + rj, 0, W - 1)
        sy = jnp.clip(BORDER + ci * CELL + ri, 0, H - 1)
        return sx, sy

    sx1, sy1 = coords(i1)
    sx2, sy2 = coords(i2)
    p1_ref[...] = sx1 * W + sy1
    pd_ref[...] = sx2 * W + sy2
    drow_ref[...] = sy2
    dcol_ref[...] = sx2
    tg1_ref[...] = ((sx1 - BORDER) % CELL) * W + sy1


def _run_argmax(m1r, r1r, m2r, r2r):
    return pl.pallas_call(
        _argmax_kernel,
        out_shape=[jax.ShapeDtypeStruct((1, N), jnp.int32)] * 5,
    )(m1r, r1r, m2r, r2r)


def _gather1_kernel(des1_ref, tgt_ref, out_ref):
    slab = des1_ref[...].reshape(D, CELL * W)
    t = tgt_ref[...].reshape(32, 1)
    onehot = (lax.broadcasted_iota(jnp.int32, (32, CELL * W), 1) == t
              ).astype(jnp.bfloat16)

    hi_mask = jnp.int32(jnp.uint32(0xFFFF0000))
    h1 = lax.bitcast_convert_type(
        lax.bitcast_convert_type(slab, jnp.int32) & hi_mask, jnp.float32)
    r1 = slab - h1
    h2 = lax.bitcast_convert_type(
        lax.bitcast_convert_type(r1, jnp.int32) & hi_mask, jnp.float32)
    h3 = r1 - h2

    def dg(m):
        return lax.dot_general(onehot, m.astype(jnp.bfloat16),
                               (((1,), (1,)), ((), ())),
                               preferred_element_type=jnp.float32)

    out_ref[...] = ((dg(h1) + dg(h2)) + dg(h3))[None]


def _run_gather1(des1_3d, tgt3):
    return pl.pallas_call(
        _gather1_kernel,
        grid=(NC,),
        in_specs=[
            pl.BlockSpec((D, CELL, W), lambda i: (0, i + 1, 0)),
            pl.BlockSpec((1, 1, 32), lambda i: (i, 0, 0)),
        ],
        out_specs=pl.BlockSpec((1, 32, D), lambda i: (i, 0, 0)),
        out_shape=jax.ShapeDtypeStruct((NC, 32, D), jnp.float32),
    )(des1_3d, tgt3)


def _sc_gather_body(p1_hbm, pd_hbm, des2t, aflowf, qlt1f, qlt2f,
                    distr_o, neigh_o, qc_o, q1_o, xx_o, yy_o, mk_o,
                    p1v, pdv, axv, ayv, q1v, xy2xv, xy2yv, maskv, qcv,
                    dsv, ngv, sem, semd):
    wid = lax.axis_index("s") * 2 + lax.axis_index("c")
    base = wid * SPT

    pltpu.sync_copy(p1_hbm.at[pl.ds(base, SPT)], p1v)
    pltpu.sync_copy(pd_hbm.at[pl.ds(base, SPT)], pdv)

    dcps = []
    cps = []
    for c in range(SPT // 16):
        sl = pl.ds(16 * c, 16)
        pch = p1v[sl]
        dcps.append(pltpu.async_copy(des2t.at[pdv[sl]], dsv.at[sl], semd))
        cps.append(pltpu.async_copy(aflowf.at[pch], axv.at[sl], sem))
        cps.append(pltpu.async_copy(aflowf.at[pch + HW], ayv.at[sl], sem))
        cps.append(pltpu.async_copy(qlt1f.at[pch], q1v.at[sl], sem))
    for cp in cps:
        cp.wait()

    qcps = []
    for c in range(SPT // 16):
        sl = pl.ds(16 * c, 16)
        xx = (axv[sl] + 0.5).astype(jnp.int32)
        yy = (ayv[sl] + 0.5).astype(jnp.int32)
        inb = (xx >= 0) & (xx < W) & (yy >= 0) & (yy < H)
        xy2xv[sl] = xx
        xy2yv[sl] = yy
        maskv[sl] = jnp.where(inb, 1, 0)
        for k, (oi, oj) in enumerate(_OFFS):
            nx = jnp.clip(xx + oi, 0, W - 1)
            ny = jnp.clip(yy + oj, 0, H - 1)
            pn = ny * W + nx
            qcps.append(pltpu.async_copy(qlt2f.at[pn], qcv.at[k, sl], sem))
            dcps.append(pltpu.async_copy(des2t.at[pn], ngv.at[k, sl], semd))
    for cp in qcps:
        cp.wait()

    pltpu.sync_copy(q1v, q1_o.at[pl.ds(base, SPT)])
    pltpu.sync_copy(xy2xv, xx_o.at[pl.ds(base, SPT)])
    pltpu.sync_copy(xy2yv, yy_o.at[pl.ds(base, SPT)])
    pltpu.sync_copy(maskv, mk_o.at[pl.ds(base, SPT)])
    pltpu.sync_copy(qcv, qc_o.at[wid])

    for cp in dcps:
        cp.wait()
    pltpu.sync_copy(dsv, distr_o.at[pl.ds(base, SPT)])
    pltpu.sync_copy(ngv, neigh_o.at[:, pl.ds(base, SPT)])


@functools.cache
def _build_sc_gather():
    return pl.kernel(
        _sc_gather_body,
        out_type=[
            jax.ShapeDtypeStruct((NP, D), jnp.float32),
            jax.ShapeDtypeStruct((K, NP, D), jnp.float32),
            jax.ShapeDtypeStruct((NTILES, 16, SPT), jnp.float32),
            jax.ShapeDtypeStruct((NP,), jnp.float32),
            jax.ShapeDtypeStruct((NP,), jnp.int32),
            jax.ShapeDtypeStruct((NP,), jnp.int32),
            jax.ShapeDtypeStruct((NP,), jnp.int32),
        ],
        mesh=_sc_mesh(),
        scratch_types=[
            pltpu.VMEM((SPT,), jnp.int32),
            pltpu.VMEM((SPT,), jnp.int32),
            pltpu.VMEM((SPT,), jnp.float32),
            pltpu.VMEM((SPT,), jnp.float32),
            pltpu.VMEM((SPT,), jnp.float32),
            pltpu.VMEM((SPT,), jnp.int32),
            pltpu.VMEM((SPT,), jnp.int32),
            pltpu.VMEM((SPT,), jnp.int32),
            pltpu.VMEM((16, SPT), jnp.float32),
            pltpu.VMEM((SPT, D), jnp.float32),
            pltpu.VMEM((K, SPT, D), jnp.float32),
            pltpu.SemaphoreType.DMA,
            pltpu.SemaphoreType.DMA,
        ],
    )


def _score_kernel(sdes_ref, neigh_ref, distr_ref, qc_ref, q1r_ref,
                  xxr_ref, yyr_ref, drow_ref, dcol_ref,
                  sc_ref, qlt_ref):
    sdes = sdes_ref[...]
    qc = jnp.transpose(qc_ref[...], (0, 2, 1)).reshape(NP, 16)
    xxc = jnp.transpose(xxr_ref[...], (1, 0))
    yyc = jnp.transpose(yyr_ref[...], (1, 0))
    q1c = jnp.transpose(q1r_ref[...], (1, 0))
    mx = jnp.full((NP, 1), -jnp.inf, jnp.float32)
    qsel = jnp.zeros((NP, 1), jnp.float32)
    for k in range(K):
        s = jnp.sum(sdes * neigh_ref[k], axis=-1, keepdims=True)
        better = s > mx
        mx = jnp.where(better, s, mx)
        qsel = jnp.where(better, qc[:, k:k + 1], qsel)
    qlt_ref[...] = ((q1c + qsel) * 0.5)[:N]

    mm = lax.dot_general(sdes, distr_ref[...],
                         (((1,), (1,)), ((), ())),
                         preferred_element_type=jnp.float32)
    dx = drow_ref[...] - xxc
    dy = dcol_ref[...] - yyc
    dis2 = dx * dx + dy * dy
    mm = jnp.where(dis2 < POS_R ** 2, 0.0, mm)
    sc_ref[:, 0:1] = mx[:N]
    sc_ref[:, 1:N + 1] = mm[:N, :N]


def _run_score(sdes, neigh, distr, qc, q1c, xxc, yyc, drowr, dcolr):
    return pl.pallas_call(
        _score_kernel,
        out_shape=[
            jax.ShapeDtypeStruct((N, N + 1), jnp.float32),
            jax.ShapeDtypeStruct((N, 1), jnp.float32),
        ],
    )(sdes, neigh, distr, qc, q1c, xxc, yyc, drowr, dcolr)


def _regroup(cm):
    t = cm.reshape(4, NC, NC, CELL).transpose(0, 3, 1, 2).reshape(4, CELL, N)
    return (t[0], lax.bitcast_convert_type(t[1], jnp.int32),
            t[2], lax.bitcast_convert_type(t[3], jnp.int32))


def kernel(des1, det1, qlt1, des2, det2, qlt2, aflow):
    des2t = des2.reshape(D, HW).T

    d1c = det1[0, 0, BORDER:H - BORDER, BORDER:W - BORDER]
    d2c = det2[0, 0, BORDER:H - BORDER, BORDER:W - BORDER]
    cm = _run_colmax(d1c, d2c)
    m1r, r1r, m2r, r2r = _regroup(cm)
    p1, pd, drow, dcol, tg1 = _run_argmax(m1r, r1r, m2r, r2r)

    tgt3 = jnp.pad(tg1.reshape(NC, NC).T, ((0, 0), (0, 2)),
                   constant_values=-1).reshape(NC, 1, 32)
    sg = _run_gather1(des1.reshape(D, H, W), tgt3)
    sdes = jnp.pad(sg[:, :NC, :].transpose(1, 0, 2).reshape(N, D),
                   ((0, NP - N), (0, 0)))

    des2t, sdes = lax.optimization_barrier((des2t, sdes))

    p1p = jnp.pad(p1.reshape(N), (0, NP - N))
    pdp = jnp.pad(pd.reshape(N), (0, NP - N))
    distr, neigh, qc, q1, xx, yy, mk = _build_sc_gather()(
        p1p, pdp, des2t,
        aflow.reshape(-1), qlt1.reshape(-1), qlt2.reshape(-1))

    drowp = jnp.pad(drow, ((0, 0), (0, NP - N)))
    dcolp = jnp.pad(dcol, ((0, 0), (0, NP - N)))
    scores, qlt = _run_score(
        sdes, neigh, distr, qc, q1.reshape(1, NP),
        xx.reshape(1, NP), yy.reshape(1, NP), drowp, dcolp)

    labels = lax.broadcasted_iota(jnp.int32, (N, N + 1), 1) == 0
    mask = (mk[:N] != 0).reshape(1, N)
    return scores, labels, mask, qlt


# speedup vs baseline: 1.0662x; 1.0662x over previous
import functools

import jax
import jax.numpy as jnp
from jax import lax
from jax.experimental import pallas as pl
from jax.experimental.pallas import tpu as pltpu
from jax.experimental.pallas import tpu_sc as plsc

H = W = 512
HW = H * W
D = 128
CELL = 16
BORDER = 16
NC = 30
N = NC * NC
NP = 1024
POS_R = 2

_OFFS = [(i, j) for i in range(-POS_R, POS_R + 1)
         for j in range(-POS_R, POS_R + 1) if i * i + j * j <= POS_R ** 2]
K = len(_OFFS)

NTILES = 32
SPT = NP // NTILES


def _sc_mesh():
    return plsc.VectorSubcoreMesh(core_axis_name="c", subcore_axis_name="s",
                                  num_cores=2, num_subcores=16)


def _colmax_kernel(d1_ref, d2_ref, out_ref):
    io = lax.broadcasted_iota(jnp.int32, (NC, CELL, NC * CELL), 1)
    res = []
    for dref in (d1_ref, d2_ref):
        x = dref[...].reshape(NC, CELL, NC * CELL)
        m = jnp.max(x, axis=1)
        r = jnp.min(jnp.where(x >= m[:, None, :], io, CELL), axis=1)
        res += [m, lax.bitcast_convert_type(r, jnp.float32)]
    out_ref[...] = jnp.stack(res, axis=0)


def _run_colmax(d1c, d2c):
    return pl.pallas_call(
        _colmax_kernel,
        out_shape=jax.ShapeDtypeStruct((4, NC, NC * CELL), jnp.float32),
    )(d1c, d2c)


def _argmax_kernel(m1_ref, r1_ref, m2_ref, r2_ref,
                   p1_ref, pd_ref, drow_ref, dcol_ref, tg1_ref):
    lane = lax.broadcasted_iota(jnp.int32, (1, NP), 1)
    ci = lane // NC
    cj = lane % NC
    rjio = lax.broadcasted_iota(jnp.int32, (CELL, NP), 0)

    def cell_argmax(m_ref, r_ref):
        m = m_ref[...]
        vmax = jnp.max(m, axis=0, keepdims=True)
        chan = r_ref[...] * CELL + rjio
        return jnp.min(jnp.where(m >= vmax, chan, 512), axis=0, keepdims=True)

    i1 = cell_argmax(m1_ref, r1_ref)
    i2 = cell_argmax(m2_ref, r2_ref)

    def coords(i):
        ri = i // CELL
        rj = i % CELL
        sx = jnp.clip(BORDER + cj * CELL + rj, 0, W - 1)
        sy = jnp.clip(BORDER + ci * CELL + ri, 0, H - 1)
        return sx, sy

    sx1, sy1 = coords(i1)
    sx2, sy2 = coords(i2)
    p1_ref[...] = sx1 * W + sy1
    pd_ref[...] = sx2 * W + sy2
    drow_ref[...] = sy2
    dcol_ref[...] = sx2
    tg1_ref[...] = ((sx1 - BORDER) % CELL) * W + sy1


def _run_argmax(m1r, r1r, m2r, r2r):
    return pl.pallas_call(
        _argmax_kernel,
        out_shape=[jax.ShapeDtypeStruct((1, NP), jnp.int32)] * 5,
    )(m1r, r1r, m2r, r2r)


def _gather1_kernel(des1_ref, tgt_ref, out_ref):
    slab = des1_ref[...].reshape(D, CELL * W)
    t = tgt_ref[...].reshape(32, 1)
    onehot = (lax.broadcasted_iota(jnp.int32, (32, CELL * W), 1) == t
              ).astype(jnp.bfloat16)

    hi_mask = jnp.int32(jnp.uint32(0xFFFF0000))
    h1 = lax.bitcast_convert_type(
        lax.bitcast_convert_type(slab, jnp.int32) & hi_mask, jnp.float32)
    r1 = slab - h1
    h2 = lax.bitcast_convert_type(
        lax.bitcast_convert_type(r1, jnp.int32) & hi_mask, jnp.float32)
    h3 = r1 - h2

    def dg(m):
        return lax.dot_general(onehot, m.astype(jnp.bfloat16),
                               (((1,), (1,)), ((), ())),
                               preferred_element_type=jnp.float32)

    out_ref[...] = ((dg(h1) + dg(h2)) + dg(h3))[None]


def _run_gather1(des1_3d, tgt3):
    return pl.pallas_call(
        _gather1_kernel,
        grid=(NC,),
        in_specs=[
            pl.BlockSpec((D, CELL, W), lambda i: (0, i + 1, 0)),
            pl.BlockSpec((1, 1, 32), lambda i: (i, 0, 0)),
        ],
        out_specs=pl.BlockSpec((1, 32, D), lambda i: (i, 0, 0)),
        out_shape=jax.ShapeDtypeStruct((NC, 32, D), jnp.float32),
    )(des1_3d, tgt3)


def _sc_gather_body(p1_hbm, pd_hbm, des2t, aflowf, qlt1f, qlt2f,
                    distr_o, neigh_o, qc_o, q1_o, xx_o, yy_o, mk_o,
                    p1v, pdv, axv, ayv, q1v, xy2xv, xy2yv, maskv, qcv,
                    dsv, ngv, sem, semd):
    wid = lax.axis_index("s") * 2 + lax.axis_index("c")
    base = wid * SPT

    pltpu.sync_copy(p1_hbm.at[pl.ds(base, SPT)], p1v)
    pltpu.sync_copy(pd_hbm.at[pl.ds(base, SPT)], pdv)

    dcps = []
    cps = []
    for c in range(SPT // 16):
        sl = pl.ds(16 * c, 16)
        pch = p1v[sl]
        dcps.append(pltpu.async_copy(des2t.at[pdv[sl]], dsv.at[sl], semd))
        cps.append(pltpu.async_copy(aflowf.at[pch], axv.at[sl], sem))
        cps.append(pltpu.async_copy(aflowf.at[pch + HW], ayv.at[sl], sem))
        cps.append(pltpu.async_copy(qlt1f.at[pch], q1v.at[sl], sem))
    for cp in cps:
        cp.wait()

    qcps = []
    for c in range(SPT // 16):
        sl = pl.ds(16 * c, 16)
        xx = (axv[sl] + 0.5).astype(jnp.int32)
        yy = (ayv[sl] + 0.5).astype(jnp.int32)
        inb = (xx >= 0) & (xx < W) & (yy >= 0) & (yy < H)
        xy2xv[sl] = xx
        xy2yv[sl] = yy
        maskv[sl] = jnp.where(inb, 1, 0)
        for k, (oi, oj) in enumerate(_OFFS):
            nx = jnp.clip(xx + oi, 0, W - 1)
            ny = jnp.clip(yy + oj, 0, H - 1)
            pn = ny * W + nx
            qcps.append(pltpu.async_copy(qlt2f.at[pn], qcv.at[k, sl], sem))
            dcps.append(pltpu.async_copy(des2t.at[pn], ngv.at[k, sl], semd))
    for cp in qcps:
        cp.wait()

    pltpu.sync_copy(q1v, q1_o.at[pl.ds(base, SPT)])
    pltpu.sync_copy(xy2xv, xx_o.at[pl.ds(base, SPT)])
    pltpu.sync_copy(xy2yv, yy_o.at[pl.ds(base, SPT)])
    pltpu.sync_copy(maskv, mk_o.at[pl.ds(base, SPT)])
    pltpu.sync_copy(qcv, qc_o.at[wid])

    for cp in dcps:
        cp.wait()
    pltpu.sync_copy(dsv, distr_o.at[pl.ds(base, SPT)])
    pltpu.sync_copy(ngv, neigh_o.at[:, pl.ds(base, SPT)])


@functools.cache
def _build_sc_gather():
    return pl.kernel(
        _sc_gather_body,
        out_type=[
            jax.ShapeDtypeStruct((NP, D), jnp.float32),
            jax.ShapeDtypeStruct((K, NP, D), jnp.float32),
            jax.ShapeDtypeStruct((NTILES, 16, SPT), jnp.float32),
            jax.ShapeDtypeStruct((NP,), jnp.float32),
            jax.ShapeDtypeStruct((NP,), jnp.int32),
            jax.ShapeDtypeStruct((NP,), jnp.int32),
            jax.ShapeDtypeStruct((NP,), jnp.int32),
        ],
        mesh=_sc_mesh(),
        scratch_types=[
            pltpu.VMEM((SPT,), jnp.int32),
            pltpu.VMEM((SPT,), jnp.int32),
            pltpu.VMEM((SPT,), jnp.float32),
            pltpu.VMEM((SPT,), jnp.float32),
            pltpu.VMEM((SPT,), jnp.float32),
            pltpu.VMEM((SPT,), jnp.int32),
            pltpu.VMEM((SPT,), jnp.int32),
            pltpu.VMEM((SPT,), jnp.int32),
            pltpu.VMEM((16, SPT), jnp.float32),
            pltpu.VMEM((SPT, D), jnp.float32),
            pltpu.VMEM((K, SPT, D), jnp.float32),
            pltpu.SemaphoreType.DMA,
            pltpu.SemaphoreType.DMA,
        ],
    )


def _score_kernel(sdes_ref, neigh_ref, distr_ref, qc_ref, q1r_ref,
                  xxr_ref, yyr_ref, drow_ref, dcol_ref,
                  sc_ref, qlt_ref):
    sdes = sdes_ref[...]
    qc = jnp.transpose(qc_ref[...], (0, 2, 1)).reshape(NP, 16)
    xxc = jnp.transpose(xxr_ref[...], (1, 0))
    yyc = jnp.transpose(yyr_ref[...], (1, 0))
    q1c = jnp.transpose(q1r_ref[...], (1, 0))
    mx = jnp.full((NP, 1), -jnp.inf, jnp.float32)
    qsel = jnp.zeros((NP, 1), jnp.float32)
    for k in range(K):
        s = jnp.sum(sdes * neigh_ref[k], axis=-1, keepdims=True)
        better = s > mx
        mx = jnp.where(better, s, mx)
        qsel = jnp.where(better, qc[:, k:k + 1], qsel)
    qlt_ref[...] = ((q1c + qsel) * 0.5)[:N]

    mm = lax.dot_general(sdes, distr_ref[...],
                         (((1,), (1,)), ((), ())),
                         preferred_element_type=jnp.float32)
    dx = drow_ref[...] - xxc
    dy = dcol_ref[...] - yyc
    dis2 = dx * dx + dy * dy
    mm = jnp.where(dis2 < POS_R ** 2, 0.0, mm)
    sc_ref[:, 0:1] = mx[:N]
    sc_ref[:, 1:N + 1] = mm[:N, :N]


def _run_score(sdes, neigh, distr, qc, q1c, xxc, yyc, drowr, dcolr):
    return pl.pallas_call(
        _score_kernel,
        out_shape=[
            jax.ShapeDtypeStruct((N, N + 1), jnp.float32),
            jax.ShapeDtypeStruct((N, 1), jnp.float32),
        ],
    )(sdes, neigh, distr, qc, q1c, xxc, yyc, drowr, dcolr)


def _regroup(cm):
    t = cm.reshape(4, NC, NC, CELL).transpose(0, 3, 1, 2).reshape(4, CELL, N)
    m1 = jnp.pad(t[0], ((0, 0), (0, NP - N)), constant_values=-1.0)
    m2 = jnp.pad(t[2], ((0, 0), (0, NP - N)), constant_values=-1.0)
    r1 = jnp.pad(lax.bitcast_convert_type(t[1], jnp.int32),
                 ((0, 0), (0, NP - N)))
    r2 = jnp.pad(lax.bitcast_convert_type(t[3], jnp.int32),
                 ((0, 0), (0, NP - N)))
    return m1, r1, m2, r2


def kernel(des1, det1, qlt1, des2, det2, qlt2, aflow):
    des2t = des2.reshape(D, HW).T

    d1c = det1[0, 0, BORDER:H - BORDER, BORDER:W - BORDER]
    d2c = det2[0, 0, BORDER:H - BORDER, BORDER:W - BORDER]
    cm = _run_colmax(d1c, d2c)
    m1r, r1r, m2r, r2r = _regroup(cm)
    p1, pd, drow, dcol, tg1 = _run_argmax(m1r, r1r, m2r, r2r)

    tgt3 = jnp.pad(tg1[0, :N].reshape(NC, NC).T, ((0, 0), (0, 2)),
                   constant_values=-1).reshape(NC, 1, 32)
    sg = _run_gather1(des1.reshape(D, H, W), tgt3)
    sdes = jnp.pad(sg[:, :NC, :].transpose(1, 0, 2).reshape(N, D),
                   ((0, NP - N), (0, 0)))

    des2t, sdes = lax.optimization_barrier((des2t, sdes))

    distr, neigh, qc, q1, xx, yy, mk = _build_sc_gather()(
        p1.reshape(NP), pd.reshape(NP), des2t,
        aflow.reshape(-1), qlt1.reshape(-1), qlt2.reshape(-1))

    scores, qlt = _run_score(
        sdes, neigh, distr, qc, q1.reshape(1, NP),
        xx.reshape(1, NP), yy.reshape(1, NP), drow, dcol)

    labels = lax.broadcasted_iota(jnp.int32, (N, N + 1), 1) == 0
    mask = (mk[:N] != 0).reshape(1, N)
    return scores, labels, mask, qlt
